# baseline (speedup 1.0000x reference)
import functools

import jax
import jax.numpy as jnp
import numpy as np
from jax.experimental import pallas as pl
from jax.experimental.pallas import tpu as pltpu

N = 16384
D = 256
H = 128
O = 128
E = 4
BN_S = 1.0 / np.sqrt(1.0 + 1e-5)

TN = 512


def _dot_t(a, b):
    return jax.lax.dot_general(a, b, (((1,), (1,)), ((), ())),
                               preferred_element_type=jnp.float32)


def _sigmoid(t):
    return 1.0 / (1.0 + jnp.exp(-t))


def _moe_body(x_ref, xb_ref, Wg_ref, W1c_ref, b1c_ref, Weh_ref, beh_ref,
              Weo_ref, beo_ref, Wsh_ref, bsh_ref, Wso_ref, bso_ref, o_ref):
    x = x_ref[...]
    xb = xb_ref[...]

    logits = _dot_t(x, Wg_ref[...])
    idx = jax.lax.broadcasted_iota(jnp.int32, logits.shape, 1)
    m1 = jnp.max(logits, axis=-1, keepdims=True)
    i1 = jnp.min(jnp.where(logits == m1, idx, E), axis=-1, keepdims=True)
    masked = jnp.where(idx == i1, -jnp.inf, logits)
    m2 = jnp.max(masked, axis=-1, keepdims=True)
    i2 = jnp.min(jnp.where(masked == m2, idx, E), axis=-1, keepdims=True)
    z = jnp.exp(m2 - m1)
    w1 = 1.0 / (1.0 + z)
    w2 = z * w1
    w = jnp.where(idx == i1, w1, 0.0) + jnp.where(idx == i2, w2, 0.0)

    hc = jnp.maximum(_dot_t(xb, W1c_ref[...]) + b1c_ref[...], 0.0) * BN_S
    hcb = hc.astype(jnp.bfloat16)

    acc = jnp.zeros((x.shape[0], O), jnp.float32)
    for e in range(E):
        he = hcb[:, e * H:(e + 1) * H]
        h = jnp.maximum(_dot_t(he, Weh_ref[e]) + beh_ref[e], 0.0) * BN_S
        o = _sigmoid(_dot_t(h.astype(jnp.bfloat16), Weo_ref[e]) + beo_ref[e])
        acc = acc + o * w[:, e:e + 1]

    hs = hcb[:, E * H:(E + 1) * H]
    h = jnp.maximum(_dot_t(hs, Wsh_ref[...]) + bsh_ref[...], 0.0) * BN_S
    sf = _sigmoid(_dot_t(h.astype(jnp.bfloat16), Wso_ref[...]) + bso_ref[...])

    o_ref[...] = acc + sf


@functools.partial(jax.jit, static_argnames=("interpret",))
def _run(combined, Wg, We1, be1, Weh, beh, Weo, beo,
         Ws1, bs1, Wsh, bsh, Wso, bso, interpret=False):
    n_tiles = N // TN
    W1c = jnp.concatenate(
        [We1.reshape(E * H, D), Ws1], axis=0).astype(jnp.bfloat16)
    b1c = jnp.concatenate([be1.reshape(1, E * H), bs1.reshape(1, H)], axis=1)
    xb = combined.astype(jnp.bfloat16)
    Wehb = Weh.astype(jnp.bfloat16)
    Weob = Weo.astype(jnp.bfloat16)
    Wshb = Wsh.astype(jnp.bfloat16)
    Wsob = Wso.astype(jnp.bfloat16)

    full = lambda shape: pl.BlockSpec(shape, lambda i: (0,) * len(shape))
    grid_spec = pl.GridSpec(
        grid=(n_tiles,),
        in_specs=[
            pl.BlockSpec((TN, D), lambda i: (i, 0)),
            pl.BlockSpec((TN, D), lambda i: (i, 0)),
            full((E, D)),
            full(((E + 1) * H, D)), full((1, (E + 1) * H)),
            full((E, H, H)), full((E, H)),
            full((E, O, H)), full((E, O)),
            full((H, H)), full((1, H)),
            full((O, H)), full((1, O)),
        ],
        out_specs=pl.BlockSpec((TN, O), lambda i: (i, 0)),
    )
    return pl.pallas_call(
        _moe_body,
        grid_spec=grid_spec,
        out_shape=jax.ShapeDtypeStruct((N, O), jnp.float32),
        compiler_params=pltpu.CompilerParams(
            dimension_semantics=("parallel",),
        ),
        interpret=interpret,
    )(combined, xb, Wg, W1c, b1c, Wehb, beh, Weob, beo,
      Wshb, bsh.reshape(1, H), Wsob, bso.reshape(1, O))


def kernel(combined, Wg, We1, be1, Weh, beh, Weo, beo,
           Ws1, bs1, Wsh, bsh, Wso, bso):
    return _run(combined, Wg, We1, be1, Weh, beh, Weo, beo,
                Ws1, bs1, Wsh, bsh, Wso, bso)


# speedup vs baseline: 1.2093x; 1.2093x over previous
import functools

import jax
import jax.numpy as jnp
import numpy as np
from jax.experimental import pallas as pl
from jax.experimental.pallas import tpu as pltpu

N = 16384
D = 256
H = 128
O = 128
E = 4
BN_S = 1.0 / np.sqrt(1.0 + 1e-5)

TN = 512


def _dot_t(a, b):
    return jax.lax.dot_general(a, b, (((1,), (1,)), ((), ())),
                               preferred_element_type=jnp.float32)


def _sigmoid(t):
    return 1.0 / (1.0 + jnp.exp(-t))


def _moe_body(x_ref, Wg_ref, W1c_ref, b1c_ref, Weh_ref, beh_ref,
              Weo_ref, beo_ref, Wsh_ref, bsh_ref, Wso_ref, bso_ref, o_ref):
    x = x_ref[...]

    logits = _dot_t(x, Wg_ref[...])
    idx = jax.lax.broadcasted_iota(jnp.int32, logits.shape, 1)
    m1 = jnp.max(logits, axis=-1, keepdims=True)
    i1 = jnp.min(jnp.where(logits == m1, idx, E), axis=-1, keepdims=True)
    masked = jnp.where(idx == i1, -jnp.inf, logits)
    m2 = jnp.max(masked, axis=-1, keepdims=True)
    i2 = jnp.min(jnp.where(masked == m2, idx, E), axis=-1, keepdims=True)
    z = jnp.exp(m2 - m1)
    w1 = 1.0 / (1.0 + z)
    w2 = z * w1
    w = jnp.where(idx == i1, w1, 0.0) + jnp.where(idx == i2, w2, 0.0)

    hc = jnp.maximum(_dot_t(x, W1c_ref[...]) + b1c_ref[...], 0.0) * BN_S

    acc = jnp.zeros((x.shape[0], O), jnp.float32)
    for e in range(E):
        he = hc[:, e * H:(e + 1) * H]
        h = jnp.maximum(_dot_t(he, Weh_ref[e]) + beh_ref[e], 0.0) * BN_S
        o = _sigmoid(_dot_t(h, Weo_ref[e]) + beo_ref[e])
        acc = acc + o * w[:, e:e + 1]

    hs = hc[:, E * H:(E + 1) * H]
    h = jnp.maximum(_dot_t(hs, Wsh_ref[...]) + bsh_ref[...], 0.0) * BN_S
    sf = _sigmoid(_dot_t(h, Wso_ref[...]) + bso_ref[...])

    o_ref[...] = acc + sf


@functools.partial(jax.jit, static_argnames=("interpret",))
def _run(combined, Wg, We1, be1, Weh, beh, Weo, beo,
         Ws1, bs1, Wsh, bsh, Wso, bso, interpret=False):
    n_tiles = N // TN
    W1c = jnp.concatenate([We1.reshape(E * H, D), Ws1], axis=0)
    b1c = jnp.concatenate([be1.reshape(1, E * H), bs1.reshape(1, H)], axis=1)

    full = lambda shape: pl.BlockSpec(shape, lambda i: (0,) * len(shape))
    grid_spec = pl.GridSpec(
        grid=(n_tiles,),
        in_specs=[
            pl.BlockSpec((TN, D), lambda i: (i, 0)),
            full((E, D)),
            full(((E + 1) * H, D)), full((1, (E + 1) * H)),
            full((E, H, H)), full((E, H)),
            full((E, O, H)), full((E, O)),
            full((H, H)), full((1, H)),
            full((O, H)), full((1, O)),
        ],
        out_specs=pl.BlockSpec((TN, O), lambda i: (i, 0)),
    )
    return pl.pallas_call(
        _moe_body,
        grid_spec=grid_spec,
        out_shape=jax.ShapeDtypeStruct((N, O), jnp.float32),
        compiler_params=pltpu.CompilerParams(
            dimension_semantics=("parallel",),
        ),
        interpret=interpret,
    )(combined, Wg, W1c, b1c, Weh, beh, Weo, beo,
      Wsh, bsh.reshape(1, H), Wso, bso.reshape(1, O))


def kernel(combined, Wg, We1, be1, Weh, beh, Weo, beo,
           Ws1, bs1, Wsh, bsh, Wso, bso):
    return _run(combined, Wg, We1, be1, Weh, beh, Weo, beo,
                Ws1, bs1, Wsh, bsh, Wso, bso)


# speedup vs baseline: 1.7083x; 1.4126x over previous
import functools

import jax
import jax.numpy as jnp
import numpy as np
from jax.experimental import pallas as pl
from jax.experimental.pallas import tpu as pltpu

N = 16384
D = 256
H = 128
O = 128
E = 4
EP = 8
BN_S = 1.0 / np.sqrt(1.0 + 1e-5)

TN = 512


def _dot_t(a, b):
    return jax.lax.dot_general(a, b, (((1,), (1,)), ((), ())),
                               preferred_element_type=jnp.float32)


def _sigmoid(t):
    return 1.0 / (1.0 + jnp.exp(-t))


def _moe_body(x_ref, Wg_ref, W1c_ref, b1c_ref, Weh_ref, beh_ref,
              Weo_ref, beo_ref, Wsh_ref, bsh_ref, Wso_ref, bso_ref, o_ref):
    x = x_ref[...]

    lt = jax.lax.dot_general(Wg_ref[...], x, (((1,), (1,)), ((), ())),
                             preferred_element_type=jnp.float32)
    eidx = jax.lax.broadcasted_iota(jnp.int32, lt.shape, 0)
    lt = jnp.where(eidx < E, lt, -jnp.inf)
    m1 = jnp.max(lt, axis=0, keepdims=True)
    i1 = jnp.min(jnp.where(lt == m1, eidx, EP), axis=0, keepdims=True)
    masked = jnp.where(eidx == i1, -jnp.inf, lt)
    m2 = jnp.max(masked, axis=0, keepdims=True)
    i2 = jnp.min(jnp.where(masked == m2, eidx, EP), axis=0, keepdims=True)
    z = jnp.exp(m2 - m1)
    w1 = 1.0 / (1.0 + z)
    w2 = z * w1
    wt = (jnp.where(eidx == i1, w1, 0.0)
          + jnp.where(eidx == i2, w2, 0.0))
    w = jnp.transpose(wt)

    hc = jnp.maximum(_dot_t(x, W1c_ref[...]) + b1c_ref[...], 0.0)

    acc = jnp.zeros((x.shape[0], O), jnp.float32)
    for e in range(E):
        he = hc[:, e * H:(e + 1) * H]
        h = jnp.maximum(_dot_t(he, Weh_ref[e]) + beh_ref[e], 0.0)
        o = _sigmoid(_dot_t(h, Weo_ref[e]) + beo_ref[e])
        acc = acc + o * w[:, e:e + 1]

    hs = hc[:, E * H:(E + 1) * H]
    h = jnp.maximum(_dot_t(hs, Wsh_ref[...]) + bsh_ref[...], 0.0)
    sf = _sigmoid(_dot_t(h, Wso_ref[...]) + bso_ref[...])

    o_ref[...] = acc + sf


@functools.partial(jax.jit, static_argnames=("interpret",))
def _run(combined, Wg, We1, be1, Weh, beh, Weo, beo,
         Ws1, bs1, Wsh, bsh, Wso, bso, interpret=False):
    n_tiles = N // TN
    W1c = jnp.concatenate([We1.reshape(E * H, D), Ws1], axis=0) * BN_S
    b1c = jnp.concatenate(
        [be1.reshape(1, E * H), bs1.reshape(1, H)], axis=1) * BN_S
    Wehs = Weh * BN_S
    behs = beh * BN_S
    Wshs = Wsh * BN_S
    bshs = bsh.reshape(1, H) * BN_S
    Wgp = jnp.concatenate([Wg, jnp.zeros((EP - E, D), Wg.dtype)], axis=0)

    full = lambda shape: pl.BlockSpec(shape, lambda i: (0,) * len(shape))
    grid_spec = pl.GridSpec(
        grid=(n_tiles,),
        in_specs=[
            pl.BlockSpec((TN, D), lambda i: (i, 0)),
            full((EP, D)),
            full(((E + 1) * H, D)), full((1, (E + 1) * H)),
            full((E, H, H)), full((E, H)),
            full((E, O, H)), full((E, O)),
            full((H, H)), full((1, H)),
            full((O, H)), full((1, O)),
        ],
        out_specs=pl.BlockSpec((TN, O), lambda i: (i, 0)),
    )
    return pl.pallas_call(
        _moe_body,
        grid_spec=grid_spec,
        out_shape=jax.ShapeDtypeStruct((N, O), jnp.float32),
        compiler_params=pltpu.CompilerParams(
            dimension_semantics=("parallel",),
        ),
        interpret=interpret,
    )(combined, Wgp, W1c, b1c, Wehs, behs, Weo, beo,
      Wshs, bshs, Wso, bso.reshape(1, O))


def kernel(combined, Wg, We1, be1, Weh, beh, Weo, beo,
           Ws1, bs1, Wsh, bsh, Wso, bso):
    return _run(combined, Wg, We1, be1, Weh, beh, Weo, beo,
                Ws1, bs1, Wsh, bsh, Wso, bso)


# speedup vs baseline: 1.9216x; 1.1249x over previous
import functools

import jax
import jax.numpy as jnp
import numpy as np
from jax.experimental import pallas as pl
from jax.experimental.pallas import tpu as pltpu

N = 16384
D = 256
H = 128
O = 128
E = 4
EP = 8
BN_S = 1.0 / np.sqrt(1.0 + 1e-5)

TN = 1024


def _dot_t(a, b):
    return jax.lax.dot_general(a, b, (((1,), (1,)), ((), ())),
                               preferred_element_type=jnp.float32)


def _sigmoid(t):
    return 1.0 / (1.0 + jnp.exp(-t))


def _moe_body(x_ref, Wg_ref, W1c_ref, b1c_ref, Weh_ref, beh_ref,
              Weo_ref, beo_ref, Wsh_ref, bsh_ref, Wso_ref, bso_ref, o_ref):
    x = x_ref[...]

    lt = jax.lax.dot_general(Wg_ref[...], x, (((1,), (1,)), ((), ())),
                             preferred_element_type=jnp.float32)
    eidx = jax.lax.broadcasted_iota(jnp.int32, lt.shape, 0)
    lt = jnp.where(eidx < E, lt, -jnp.inf)
    m1 = jnp.max(lt, axis=0, keepdims=True)
    i1 = jnp.min(jnp.where(lt == m1, eidx, EP), axis=0, keepdims=True)
    masked = jnp.where(eidx == i1, -jnp.inf, lt)
    m2 = jnp.max(masked, axis=0, keepdims=True)
    i2 = jnp.min(jnp.where(masked == m2, eidx, EP), axis=0, keepdims=True)
    z = jnp.exp(m2 - m1)
    w1 = 1.0 / (1.0 + z)
    w2 = z * w1
    wt = (jnp.where(eidx == i1, w1, 0.0)
          + jnp.where(eidx == i2, w2, 0.0))
    w = jnp.transpose(wt)

    hc = jnp.maximum(_dot_t(x, W1c_ref[...]) + b1c_ref[...], 0.0)

    acc = jnp.zeros((x.shape[0], O), jnp.float32)
    for e in range(E):
        he = hc[:, e * H:(e + 1) * H]
        h = jnp.maximum(_dot_t(he, Weh_ref[e]) + beh_ref[e], 0.0)
        o = _sigmoid(_dot_t(h, Weo_ref[e]) + beo_ref[e])
        acc = acc + o * w[:, e:e + 1]

    hs = hc[:, E * H:(E + 1) * H]
    h = jnp.maximum(_dot_t(hs, Wsh_ref[...]) + bsh_ref[...], 0.0)
    sf = _sigmoid(_dot_t(h, Wso_ref[...]) + bso_ref[...])

    o_ref[...] = acc + sf


@functools.partial(jax.jit, static_argnames=("interpret",))
def _run(combined, Wg, We1, be1, Weh, beh, Weo, beo,
         Ws1, bs1, Wsh, bsh, Wso, bso, interpret=False):
    n_tiles = N // TN
    W1c = jnp.concatenate([We1.reshape(E * H, D), Ws1], axis=0) * BN_S
    b1c = jnp.concatenate(
        [be1.reshape(1, E * H), bs1.reshape(1, H)], axis=1) * BN_S
    Wehs = Weh * BN_S
    behs = beh * BN_S
    Wshs = Wsh * BN_S
    bshs = bsh.reshape(1, H) * BN_S
    Wgp = jnp.concatenate([Wg, jnp.zeros((EP - E, D), Wg.dtype)], axis=0)

    full = lambda shape: pl.BlockSpec(shape, lambda i: (0,) * len(shape))
    grid_spec = pl.GridSpec(
        grid=(n_tiles,),
        in_specs=[
            pl.BlockSpec((TN, D), lambda i: (i, 0)),
            full((EP, D)),
            full(((E + 1) * H, D)), full((1, (E + 1) * H)),
            full((E, H, H)), full((E, H)),
            full((E, O, H)), full((E, O)),
            full((H, H)), full((1, H)),
            full((O, H)), full((1, O)),
        ],
        out_specs=pl.BlockSpec((TN, O), lambda i: (i, 0)),
    )
    return pl.pallas_call(
        _moe_body,
        grid_spec=grid_spec,
        out_shape=jax.ShapeDtypeStruct((N, O), jnp.float32),
        compiler_params=pltpu.CompilerParams(
            dimension_semantics=("parallel",),
        ),
        interpret=interpret,
    )(combined, Wgp, W1c, b1c, Wehs, behs, Weo, beo,
      Wshs, bshs, Wso, bso.reshape(1, O))


def kernel(combined, Wg, We1, be1, Weh, beh, Weo, beo,
           Ws1, bs1, Wsh, bsh, Wso, bso):
    return _run(combined, Wg, We1, be1, Weh, beh, Weo, beo,
                Ws1, bs1, Wsh, bsh, Wso, bso)


# speedup vs baseline: 2.1034x; 1.0946x over previous
import functools

import jax
import jax.numpy as jnp
import numpy as np
from jax.experimental import pallas as pl
from jax.experimental.pallas import tpu as pltpu

N = 16384
D = 256
H = 128
O = 128
E = 4
EP = 8
BN_S = 1.0 / np.sqrt(1.0 + 1e-5)

TN = 2048


def _dot_t(a, b):
    return jax.lax.dot_general(a, b, (((1,), (1,)), ((), ())),
                               preferred_element_type=jnp.float32)


def _sigmoid(t):
    return 1.0 / (1.0 + jnp.exp(-t))


def _moe_body(x_ref, Wg_ref, W1c_ref, b1c_ref, Weh_ref, beh_ref,
              Weo_ref, beo_ref, Wsh_ref, bsh_ref, Wso_ref, bso_ref, o_ref):
    x = x_ref[...]

    lt = jax.lax.dot_general(Wg_ref[...], x, (((1,), (1,)), ((), ())),
                             preferred_element_type=jnp.float32)
    eidx = jax.lax.broadcasted_iota(jnp.int32, lt.shape, 0)
    lt = jnp.where(eidx < E, lt, -jnp.inf)
    m1 = jnp.max(lt, axis=0, keepdims=True)
    i1 = jnp.min(jnp.where(lt == m1, eidx, EP), axis=0, keepdims=True)
    masked = jnp.where(eidx == i1, -jnp.inf, lt)
    m2 = jnp.max(masked, axis=0, keepdims=True)
    i2 = jnp.min(jnp.where(masked == m2, eidx, EP), axis=0, keepdims=True)
    z = jnp.exp(m2 - m1)
    w1 = 1.0 / (1.0 + z)
    w2 = z * w1
    wt = (jnp.where(eidx == i1, w1, 0.0)
          + jnp.where(eidx == i2, w2, 0.0))
    w = jnp.transpose(wt)

    hc = jnp.maximum(_dot_t(x, W1c_ref[...]) + b1c_ref[...], 0.0)

    acc = jnp.zeros((x.shape[0], O), jnp.float32)
    for e in range(E):
        he = hc[:, e * H:(e + 1) * H]
        h = jnp.maximum(_dot_t(he, Weh_ref[e]) + beh_ref[e], 0.0)
        o = _sigmoid(_dot_t(h, Weo_ref[e]) + beo_ref[e])
        acc = acc + o * w[:, e:e + 1]

    hs = hc[:, E * H:(E + 1) * H]
    h = jnp.maximum(_dot_t(hs, Wsh_ref[...]) + bsh_ref[...], 0.0)
    sf = _sigmoid(_dot_t(h, Wso_ref[...]) + bso_ref[...])

    o_ref[...] = acc + sf


@functools.partial(jax.jit, static_argnames=("interpret",))
def _run(combined, Wg, We1, be1, Weh, beh, Weo, beo,
         Ws1, bs1, Wsh, bsh, Wso, bso, interpret=False):
    n_tiles = N // TN
    W1c = jnp.concatenate([We1.reshape(E * H, D), Ws1], axis=0) * BN_S
    b1c = jnp.concatenate(
        [be1.reshape(1, E * H), bs1.reshape(1, H)], axis=1) * BN_S
    Wehs = Weh * BN_S
    behs = beh * BN_S
    Wshs = Wsh * BN_S
    bshs = bsh.reshape(1, H) * BN_S
    Wgp = jnp.concatenate([Wg, jnp.zeros((EP - E, D), Wg.dtype)], axis=0)

    full = lambda shape: pl.BlockSpec(shape, lambda i: (0,) * len(shape))
    grid_spec = pl.GridSpec(
        grid=(n_tiles,),
        in_specs=[
            pl.BlockSpec((TN, D), lambda i: (i, 0)),
            full((EP, D)),
            full(((E + 1) * H, D)), full((1, (E + 1) * H)),
            full((E, H, H)), full((E, H)),
            full((E, O, H)), full((E, O)),
            full((H, H)), full((1, H)),
            full((O, H)), full((1, O)),
        ],
        out_specs=pl.BlockSpec((TN, O), lambda i: (i, 0)),
    )
    return pl.pallas_call(
        _moe_body,
        grid_spec=grid_spec,
        out_shape=jax.ShapeDtypeStruct((N, O), jnp.float32),
        compiler_params=pltpu.CompilerParams(
            dimension_semantics=("parallel",),
        ),
        interpret=interpret,
    )(combined, Wgp, W1c, b1c, Wehs, behs, Weo, beo,
      Wshs, bshs, Wso, bso.reshape(1, O))


def kernel(combined, Wg, We1, be1, Weh, beh, Weo, beo,
           Ws1, bs1, Wsh, bsh, Wso, bso):
    return _run(combined, Wg, We1, be1, Weh, beh, Weo, beo,
                Ws1, bs1, Wsh, bsh, Wso, bso)
